# Initial kernel scaffold; baseline (speedup 1.0000x reference)
import jax
import jax.numpy as jnp
from jax.experimental import pallas as pl


def kernel(x, edge_index, batch, W1, b1, W2, b2, W3, b3, convW, convB, lin1W, lin1b, lin2W, lin2b):
    raise NotImplementedError("write your pallas kernel here")


# speedup vs baseline: 8.1686x; 8.1686x over previous
import functools

import jax
import jax.numpy as jnp
from jax import lax
from jax.experimental import pallas as pl
from jax.experimental.pallas import tpu as pltpu
from jax.experimental.pallas import tpu_sc as plsc

N = 10000
HD = 128
NB = 32
KTOP = 30
CONV_OUT = 32
CONV_KER = 5
OUT_LEN = KTOP - CONV_KER + 1

NP = 10240
NC = 2
NS = 16
NW = NC * NS
ROWS_PT = NP // NS
CHUNK = 128

@functools.lru_cache(maxsize=None)
def _sc_kernels(epad):
    _mesh = plsc.VectorSubcoreMesh(core_axis_name="c", subcore_axis_name="s",
                                   num_cores=NC, num_subcores=NS)
    nchunk = epad // (NW * CHUNK)
    edges_pt = nchunk * CHUNK

    nblk = ROWS_PT // CHUNK

    @functools.partial(
        pl.kernel,
        out_type=jax.ShapeDtypeStruct((NC * NP, HD), jnp.float32),
        mesh=_mesh,
        scratch_types=[
            pltpu.VMEM((CHUNK,), jnp.int32),
            pltpu.VMEM((CHUNK,), jnp.int32),
            pltpu.VMEM((CHUNK, HD), jnp.float32),
            pltpu.VMEM_SHARED((NP, HD), jnp.float32),
            pltpu.SemaphoreType.DMA,
        ],
    )
    def message(y_hbm, src_hbm, dst_hbm, zeros_hbm, out_hbm,
                sidx_v, didx_v, rows_v, acc_sh, sem):
        cid = lax.axis_index("c")
        sid = lax.axis_index("s")
        wid = cid * NS + sid
        r0 = sid * ROWS_PT
        pltpu.sync_copy(zeros_hbm.at[pl.ds(0, CHUNK)], rows_v)
        for k in range(nblk):
            pltpu.sync_copy(rows_v, acc_sh.at[pl.ds(r0 + k * CHUNK, CHUNK)])
        plsc.subcore_barrier()
        ebase = wid * edges_pt

        def body(i, carry):
            b = ebase + i * CHUNK
            pltpu.sync_copy(src_hbm.at[pl.ds(b, CHUNK)], sidx_v)
            pltpu.sync_copy(dst_hbm.at[pl.ds(b, CHUNK)], didx_v)
            pltpu.sync_copy(y_hbm.at[sidx_v], rows_v)
            pltpu.sync_copy(rows_v, acc_sh.at[didx_v], add=True)
            return carry

        lax.fori_loop(0, nchunk, body, 0)
        plsc.subcore_barrier()
        for k in range(nblk):
            pltpu.sync_copy(acc_sh.at[pl.ds(r0 + k * CHUNK, CHUNK)], rows_v)
            pltpu.sync_copy(rows_v,
                            out_hbm.at[pl.ds(cid * NP + r0 + k * CHUNK, CHUNK)])

    return message


RB = 640


def _row_valid(rb):
    i = pl.program_id(0)
    grow = i * rb + lax.broadcasted_iota(jnp.int32, (rb, 1), 0)
    return grow < N


def _tc_pre_body(x_ref, w_ref, deg_ref, y_ref, dinv_ref):
    dv = lax.rsqrt(deg_ref[0] + deg_ref[1])
    xw = jnp.dot(x_ref[...], w_ref[...], preferred_element_type=jnp.float32)
    valid = _row_valid(RB)
    y_ref[...] = jnp.where(valid, xw * dv, 0.0)
    dinv_ref[...] = dv


def _tc_mid_body(acc_ref, dinv_ref, b_ref, w_ref, y_ref):
    dv = dinv_ref[...]
    h = jnp.maximum(dv * (acc_ref[0] + acc_ref[1]) + b_ref[...], 0.0)
    y = jnp.dot(h, w_ref[...], preferred_element_type=jnp.float32) * dv
    y_ref[...] = jnp.where(_row_valid(RB), y, 0.0)


def _tc_fin_body(acc_ref, dinv_ref, b_ref, h_ref):
    dv = dinv_ref[...]
    h = jnp.maximum(dv * (acc_ref[0] + acc_ref[1]) + b_ref[...], 0.0)
    h_ref[...] = jnp.where(_row_valid(RB), h, 0.0)


def _tc_head_body(h_ref, k2_ref, b2_ref, cw_ref, cb_ref, l1_ref, l1b_ref,
                  l2_ref, l2b_ref, out_ref, topk_ref, flat_ref):
    keys = k2_ref[...]
    bt = b2_ref[...]
    flatpos = (lax.broadcasted_iota(jnp.int32, (NP // 128, 128), 0) * 128
               + lax.broadcasted_iota(jnp.int32, (NP // 128, 128), 1))
    for g in range(NB):
        kg0 = jnp.where(bt == g, keys, -1.0)

        def sel(k, kg, g=g):
            m = jnp.max(kg)
            pos = jnp.min(jnp.where(kg == m, flatpos, jnp.int32(2 ** 30)))
            pos = jnp.minimum(pos, NP - 1)
            valid = jnp.where(m >= 0.0, 1.0, 0.0)
            topk_ref[g, pl.ds(k, 1), :] = h_ref[pl.ds(pos, 1), :] * valid
            return jnp.where(flatpos == pos, -2.0, kg)

        lax.fori_loop(0, KTOP, sel, kg0)

    topk = topk_ref[...]
    cw = cw_ref[...]
    cb = cb_ref[...]
    for t in range(OUT_LEN):
        s = jnp.zeros((NB, CONV_OUT), jnp.float32)
        for kk in range(CONV_KER):
            s = s + jnp.dot(topk[:, t + kk, :], cw[kk],
                            preferred_element_type=jnp.float32)
        flat_ref[:, pl.ds(t * CONV_OUT, CONV_OUT)] = jnp.maximum(s + cb, 0.0)
    flat = flat_ref[...]
    h1 = jnp.maximum(
        jnp.dot(flat, l1_ref[...], preferred_element_type=jnp.float32)
        + l1b_ref[...], 0.0)
    out_ref[...] = (jnp.dot(h1, l2_ref[...], preferred_element_type=jnp.float32)
                    + l2b_ref[...])


_f32 = jnp.float32
_spec_rows = pl.BlockSpec((RB, HD), lambda i: (i, 0))
_spec_acc = pl.BlockSpec((2, RB, HD), lambda i: (0, i, 0))
_spec_col = pl.BlockSpec((RB, 1), lambda i: (i, 0))
_spec_w = pl.BlockSpec((HD, HD), lambda i: (0, 0))
_spec_b = pl.BlockSpec((1, HD), lambda i: (0, 0))

_tc_pre = pl.pallas_call(
    _tc_pre_body,
    grid=(NP // RB,),
    in_specs=[_spec_rows, _spec_w, pl.BlockSpec((2, RB, 1), lambda i: (0, i, 0))],
    out_specs=[_spec_rows, _spec_col],
    out_shape=[jax.ShapeDtypeStruct((NP, HD), _f32),
               jax.ShapeDtypeStruct((NP, 1), _f32)],
)

_tc_mid = pl.pallas_call(
    _tc_mid_body,
    grid=(NP // RB,),
    in_specs=[_spec_acc, _spec_col, _spec_b, _spec_w],
    out_specs=_spec_rows,
    out_shape=jax.ShapeDtypeStruct((NP, HD), _f32),
)

_tc_fin = pl.pallas_call(
    _tc_fin_body,
    grid=(NP // RB,),
    in_specs=[_spec_acc, _spec_col, _spec_b],
    out_specs=_spec_rows,
    out_shape=jax.ShapeDtypeStruct((NP, HD), _f32),
)

_tc_head = pl.pallas_call(
    _tc_head_body,
    out_shape=jax.ShapeDtypeStruct((NB, HD), _f32),
    scratch_shapes=[pltpu.VMEM((NB, KTOP, HD), _f32),
                    pltpu.VMEM((NB, OUT_LEN * CONV_OUT), _f32)],
)


def kernel(x, edge_index, batch, W1, b1, W2, b2, W3, b3, convW, convB,
           lin1W, lin1b, lin2W, lin2b):
    e = edge_index.astype(jnp.int32)
    loop = jnp.arange(N, dtype=jnp.int32)
    src = jnp.concatenate([e[0], loop])
    dst = jnp.concatenate([e[1], loop])
    ne = src.shape[0]
    epad = -(-ne // (NW * CHUNK)) * (NW * CHUNK)
    src = jnp.concatenate([src, jnp.full((epad - ne,), N, jnp.int32)])
    dst = jnp.concatenate([dst, jnp.full((epad - ne,), N, jnp.int32)])

    sc_message = _sc_kernels(epad)

    z128 = jnp.zeros((NP, HD), _f32)
    ones128 = jnp.ones((NP, HD), _f32)
    xpad = jnp.pad(x, ((0, NP - N), (0, 0)))
    b1r, b2r, b3r = (b.reshape(1, HD) for b in (b1, b2, b3))

    deg = sc_message(ones128, dst, dst, z128).reshape(NC, NP, HD)
    deg2 = deg[:, :, 0:1]

    y1, dinv = _tc_pre(xpad, W1, deg2)
    acc1 = sc_message(y1, src, dst, z128).reshape(NC, NP, HD)
    y2 = _tc_mid(acc1, dinv, b1r, W2)
    acc2 = sc_message(y2, src, dst, z128).reshape(NC, NP, HD)
    y3 = _tc_mid(acc2, dinv, b2r, W3)
    acc3 = sc_message(y3, src, dst, z128).reshape(NC, NP, HD)
    h3 = _tc_fin(acc3, dinv, b3r)

    keys2d = h3[:, HD - 1].reshape(NP // 128, 128)
    batch2d = jnp.pad(batch.astype(jnp.int32), (0, NP - N),
                      constant_values=-1).reshape(NP // 128, 128)

    cwt = jnp.transpose(convW, (2, 1, 0))
    cbr = convB.reshape(1, CONV_OUT)
    l1p = jnp.transpose(lin1W.reshape(HD, CONV_OUT, OUT_LEN),
                        (2, 1, 0)).reshape(OUT_LEN * CONV_OUT, HD)
    l1br = lin1b.reshape(1, HD)
    l2p = jnp.zeros((HD, HD), _f32).at[:, :lin2W.shape[0]].set(lin2W.T)
    l2br = jnp.zeros((1, HD), _f32).at[:, :lin2b.shape[0]].set(lin2b)

    logits_pad = _tc_head(h3, keys2d, batch2d, cwt, cbr, l1p, l1br, l2p, l2br)
    return (logits_pad[:, :lin2W.shape[0]], jnp.zeros((), _f32))


# speedup vs baseline: 8.8214x; 1.0799x over previous
import functools

import jax
import jax.numpy as jnp
from jax import lax
from jax.experimental import pallas as pl
from jax.experimental.pallas import tpu as pltpu
from jax.experimental.pallas import tpu_sc as plsc

N = 10000
HD = 128
NB = 32
KTOP = 30
CONV_OUT = 32
CONV_KER = 5
OUT_LEN = KTOP - CONV_KER + 1

NP = 10240
NC = 2
NS = 16
NW = NC * NS
ROWS_PT = NP // NS
CHUNK = 128

@functools.lru_cache(maxsize=None)
def _sc_kernels(epad):
    _mesh = plsc.VectorSubcoreMesh(core_axis_name="c", subcore_axis_name="s",
                                   num_cores=NC, num_subcores=NS)
    nchunk = epad // (NW * CHUNK)
    edges_pt = nchunk * CHUNK

    nblk = ROWS_PT // CHUNK

    @functools.partial(
        pl.kernel,
        out_type=jax.ShapeDtypeStruct((NC * NP, HD), jnp.float32),
        mesh=_mesh,
        scratch_types=[
            pltpu.VMEM((CHUNK,), jnp.int32),
            pltpu.VMEM((CHUNK,), jnp.int32),
            pltpu.VMEM((CHUNK,), jnp.int32),
            pltpu.VMEM((CHUNK,), jnp.int32),
            pltpu.VMEM((CHUNK, HD), jnp.float32),
            pltpu.VMEM((CHUNK, HD), jnp.float32),
            pltpu.VMEM_SHARED((NP, HD), jnp.float32),
            pltpu.SemaphoreType.DMA,
            pltpu.SemaphoreType.DMA,
        ],
    )
    def message(y_hbm, src_hbm, dst_hbm, zeros_hbm, out_hbm,
                s0, s1, d0, d1, r0v, r1v, acc_sh, sem0, sem1):
        cid = lax.axis_index("c")
        sid = lax.axis_index("s")
        wid = cid * NS + sid
        r0 = sid * ROWS_PT
        pltpu.sync_copy(zeros_hbm.at[pl.ds(0, CHUNK)], r0v)
        for k in range(nblk):
            pltpu.sync_copy(r0v, acc_sh.at[pl.ds(r0 + k * CHUNK, CHUNK)])
        plsc.subcore_barrier()
        ebase = wid * edges_pt

        pltpu.sync_copy(src_hbm.at[pl.ds(ebase, CHUNK)], s0)
        pltpu.async_copy(y_hbm.at[s0], r0v, sem0)
        pltpu.sync_copy(dst_hbm.at[pl.ds(ebase, CHUNK)], d0)

        def body(j, carry):
            b = ebase + (2 * j + 1) * CHUNK
            nxt = ebase + (2 * j + 2) * CHUNK
            pltpu.sync_copy(src_hbm.at[pl.ds(b, CHUNK)], s1)
            pltpu.async_copy(y_hbm.at[s1], r1v, sem1)
            pltpu.sync_copy(dst_hbm.at[pl.ds(b, CHUNK)], d1)
            pltpu.make_async_copy(y_hbm.at[pl.ds(0, CHUNK)], r0v, sem0).wait()
            pltpu.sync_copy(r0v, acc_sh.at[d0], add=True)

            @pl.when(2 * j + 2 < nchunk)
            def _():
                pltpu.sync_copy(src_hbm.at[pl.ds(nxt, CHUNK)], s0)
                pltpu.async_copy(y_hbm.at[s0], r0v, sem0)
                pltpu.sync_copy(dst_hbm.at[pl.ds(nxt, CHUNK)], d0)

            pltpu.make_async_copy(y_hbm.at[pl.ds(0, CHUNK)], r1v, sem1).wait()
            pltpu.sync_copy(r1v, acc_sh.at[d1], add=True)
            return carry

        lax.fori_loop(0, nchunk // 2, body, 0)
        plsc.subcore_barrier()
        for k in range(nblk):
            pltpu.sync_copy(acc_sh.at[pl.ds(r0 + k * CHUNK, CHUNK)], r0v)
            pltpu.sync_copy(r0v,
                            out_hbm.at[pl.ds(cid * NP + r0 + k * CHUNK, CHUNK)])

    @functools.partial(
        pl.kernel,
        out_type=jax.ShapeDtypeStruct((NC * NP, HD), jnp.float32),
        mesh=_mesh,
        scratch_types=[
            pltpu.VMEM((CHUNK,), jnp.int32),
            pltpu.VMEM((CHUNK, HD), jnp.float32),
            pltpu.VMEM_SHARED((NP, HD), jnp.float32),
            pltpu.SemaphoreType.DMA,
        ],
    )
    def spread(dst_hbm, ones_hbm, zeros_hbm, out_hbm, didx_v, rows_v, acc_sh, sem):
        cid = lax.axis_index("c")
        sid = lax.axis_index("s")
        wid = cid * NS + sid
        r0 = sid * ROWS_PT
        pltpu.sync_copy(zeros_hbm.at[pl.ds(0, CHUNK)], rows_v)
        for k in range(nblk):
            pltpu.sync_copy(rows_v, acc_sh.at[pl.ds(r0 + k * CHUNK, CHUNK)])
        pltpu.sync_copy(ones_hbm.at[pl.ds(0, CHUNK)], rows_v)
        plsc.subcore_barrier()
        ebase = wid * edges_pt

        def body(i, carry):
            b = ebase + i * CHUNK
            pltpu.sync_copy(dst_hbm.at[pl.ds(b, CHUNK)], didx_v)
            pltpu.sync_copy(rows_v, acc_sh.at[didx_v], add=True)
            return carry

        lax.fori_loop(0, nchunk, body, 0)
        plsc.subcore_barrier()
        for k in range(nblk):
            pltpu.sync_copy(acc_sh.at[pl.ds(r0 + k * CHUNK, CHUNK)], rows_v)
            pltpu.sync_copy(rows_v,
                            out_hbm.at[pl.ds(cid * NP + r0 + k * CHUNK, CHUNK)])

    return message, spread


RB = 640


def _row_valid(rb):
    i = pl.program_id(0)
    grow = i * rb + lax.broadcasted_iota(jnp.int32, (rb, 1), 0)
    return grow < N


def _tc_pre_body(x_ref, w_ref, deg_ref, y_ref, dinv_ref):
    dv = lax.rsqrt(deg_ref[0] + deg_ref[1])
    xw = jnp.dot(x_ref[...], w_ref[...], preferred_element_type=jnp.float32)
    valid = _row_valid(RB)
    y_ref[...] = jnp.where(valid, xw * dv, 0.0)
    dinv_ref[...] = dv


def _tc_mid_body(acc_ref, dinv_ref, b_ref, w_ref, y_ref):
    dv = dinv_ref[...]
    h = jnp.maximum(dv * (acc_ref[0] + acc_ref[1]) + b_ref[...], 0.0)
    y = jnp.dot(h, w_ref[...], preferred_element_type=jnp.float32) * dv
    y_ref[...] = jnp.where(_row_valid(RB), y, 0.0)


def _tc_fin_body(acc_ref, dinv_ref, b_ref, h_ref):
    dv = dinv_ref[...]
    h = jnp.maximum(dv * (acc_ref[0] + acc_ref[1]) + b_ref[...], 0.0)
    h_ref[...] = jnp.where(_row_valid(RB), h, 0.0)


def _tc_head_body(h_ref, k2_ref, b2_ref, cw_ref, cb_ref, l1_ref, l1b_ref,
                  l2_ref, l2b_ref, out_ref, topk_ref, flat_ref):
    keys = k2_ref[...]
    bt = b2_ref[...]
    flatpos = (lax.broadcasted_iota(jnp.int32, (NP // 128, 128), 0) * 128
               + lax.broadcasted_iota(jnp.int32, (NP // 128, 128), 1))
    for g in range(NB):
        kg0 = jnp.where(bt == g, keys, -1.0)

        def sel(k, kg, g=g):
            m = jnp.max(kg)
            pos = jnp.min(jnp.where(kg == m, flatpos, jnp.int32(2 ** 30)))
            pos = jnp.minimum(pos, NP - 1)
            valid = jnp.where(m >= 0.0, 1.0, 0.0)
            topk_ref[g, pl.ds(k, 1), :] = h_ref[pl.ds(pos, 1), :] * valid
            return jnp.where(flatpos == pos, -2.0, kg)

        lax.fori_loop(0, KTOP, sel, kg0)

    topk = topk_ref[...]
    cw = cw_ref[...]
    cb = cb_ref[...]
    for t in range(OUT_LEN):
        s = jnp.zeros((NB, CONV_OUT), jnp.float32)
        for kk in range(CONV_KER):
            s = s + jnp.dot(topk[:, t + kk, :], cw[kk],
                            preferred_element_type=jnp.float32)
        flat_ref[:, pl.ds(t * CONV_OUT, CONV_OUT)] = jnp.maximum(s + cb, 0.0)
    flat = flat_ref[...]
    h1 = jnp.maximum(
        jnp.dot(flat, l1_ref[...], preferred_element_type=jnp.float32)
        + l1b_ref[...], 0.0)
    out_ref[...] = (jnp.dot(h1, l2_ref[...], preferred_element_type=jnp.float32)
                    + l2b_ref[...])


_f32 = jnp.float32
_spec_rows = pl.BlockSpec((RB, HD), lambda i: (i, 0))
_spec_acc = pl.BlockSpec((2, RB, HD), lambda i: (0, i, 0))
_spec_col = pl.BlockSpec((RB, 1), lambda i: (i, 0))
_spec_w = pl.BlockSpec((HD, HD), lambda i: (0, 0))
_spec_b = pl.BlockSpec((1, HD), lambda i: (0, 0))

_tc_pre = pl.pallas_call(
    _tc_pre_body,
    grid=(NP // RB,),
    in_specs=[_spec_rows, _spec_w, pl.BlockSpec((2, RB, 1), lambda i: (0, i, 0))],
    out_specs=[_spec_rows, _spec_col],
    out_shape=[jax.ShapeDtypeStruct((NP, HD), _f32),
               jax.ShapeDtypeStruct((NP, 1), _f32)],
)

_tc_mid = pl.pallas_call(
    _tc_mid_body,
    grid=(NP // RB,),
    in_specs=[_spec_acc, _spec_col, _spec_b, _spec_w],
    out_specs=_spec_rows,
    out_shape=jax.ShapeDtypeStruct((NP, HD), _f32),
)

_tc_fin = pl.pallas_call(
    _tc_fin_body,
    grid=(NP // RB,),
    in_specs=[_spec_acc, _spec_col, _spec_b],
    out_specs=_spec_rows,
    out_shape=jax.ShapeDtypeStruct((NP, HD), _f32),
)

_tc_head = pl.pallas_call(
    _tc_head_body,
    out_shape=jax.ShapeDtypeStruct((NB, HD), _f32),
    scratch_shapes=[pltpu.VMEM((NB, KTOP, HD), _f32),
                    pltpu.VMEM((NB, OUT_LEN * CONV_OUT), _f32)],
)


def kernel(x, edge_index, batch, W1, b1, W2, b2, W3, b3, convW, convB,
           lin1W, lin1b, lin2W, lin2b):
    e = edge_index.astype(jnp.int32)
    loop = jnp.arange(N, dtype=jnp.int32)
    src = jnp.concatenate([e[0], loop])
    dst = jnp.concatenate([e[1], loop])
    ne = src.shape[0]
    unit = NW * CHUNK * 2
    epad = -(-ne // unit) * unit
    src = jnp.concatenate([src, jnp.full((epad - ne,), N, jnp.int32)])
    dst = jnp.concatenate([dst, jnp.full((epad - ne,), N, jnp.int32)])

    sc_message, sc_spread = _sc_kernels(epad)

    z128 = jnp.zeros((NP, HD), _f32)
    ones128 = jnp.ones((NP, HD), _f32)
    xpad = jnp.pad(x, ((0, NP - N), (0, 0)))
    b1r, b2r, b3r = (b.reshape(1, HD) for b in (b1, b2, b3))

    deg = sc_spread(dst, ones128, z128).reshape(NC, NP, HD)
    deg2 = deg[:, :, 0:1]

    y1, dinv = _tc_pre(xpad, W1, deg2)
    acc1 = sc_message(y1, src, dst, z128).reshape(NC, NP, HD)
    y2 = _tc_mid(acc1, dinv, b1r, W2)
    acc2 = sc_message(y2, src, dst, z128).reshape(NC, NP, HD)
    y3 = _tc_mid(acc2, dinv, b2r, W3)
    acc3 = sc_message(y3, src, dst, z128).reshape(NC, NP, HD)
    h3 = _tc_fin(acc3, dinv, b3r)

    keys2d = h3[:, HD - 1].reshape(NP // 128, 128)
    batch2d = jnp.pad(batch.astype(jnp.int32), (0, NP - N),
                      constant_values=-1).reshape(NP // 128, 128)

    cwt = jnp.transpose(convW, (2, 1, 0))
    cbr = convB.reshape(1, CONV_OUT)
    l1p = jnp.transpose(lin1W.reshape(HD, CONV_OUT, OUT_LEN),
                        (2, 1, 0)).reshape(OUT_LEN * CONV_OUT, HD)
    l1br = lin1b.reshape(1, HD)
    l2p = jnp.zeros((HD, HD), _f32).at[:, :lin2W.shape[0]].set(lin2W.T)
    l2br = jnp.zeros((1, HD), _f32).at[:, :lin2b.shape[0]].set(lin2b)

    logits_pad = _tc_head(h3, keys2d, batch2d, cwt, cbr, l1p, l1br, l2p, l2br)
    return (logits_pad[:, :lin2W.shape[0]], jnp.zeros((), _f32))
